# Initial kernel scaffold; baseline (speedup 1.0000x reference)
import jax
import jax.numpy as jnp
from jax.experimental import pallas as pl


def kernel(node_features, src_idx, edge_features, delta_t, edge_len, params):
    raise NotImplementedError("write your pallas kernel here")


# speedup vs baseline: 7.6417x; 7.6417x over previous
import functools
import math

import jax
import jax.numpy as jnp
import numpy as np
from jax import lax
from jax.experimental import pallas as pl
from jax.experimental.pallas import tpu as pltpu
from jax.experimental.pallas import tpu_sc as plsc

H = 128
T = 4
EDGE_IN = 16
DEG = 16
NUM_CLASS = 16

_NB = 200


def _sc_gather(table, idx):
    E = idx.shape[0]
    D = table.shape[1]
    CH = 128
    n_chunks = E // CH
    assert n_chunks * CH == E
    info = plsc.get_sparse_core_info()
    nc = info.num_cores
    nw = nc * info.num_subcores
    per_w = math.ceil(n_chunks / nw)
    mesh = plsc.VectorSubcoreMesh(core_axis_name="c", subcore_axis_name="s")

    @functools.partial(
        pl.kernel,
        out_type=jax.ShapeDtypeStruct((E, D), table.dtype),
        mesh=mesh,
        scratch_types=[
            pltpu.VMEM((CH,), jnp.int32),
            pltpu.VMEM((CH, D), jnp.float32),
            pltpu.SemaphoreType.DMA,
        ],
    )
    def k(table_hbm, idx_hbm, out_hbm, idx_v, rows_v, sem):
        wid = lax.axis_index("s") * nc + lax.axis_index("c")

        def body(t, carry):
            c = wid + t * nw

            @pl.when(c < n_chunks)
            def _():
                base = c * CH
                pltpu.sync_copy(idx_hbm.at[pl.ds(base, CH)], idx_v)
                pltpu.async_copy(table_hbm.at[idx_v], rows_v, sem).wait()
                pltpu.sync_copy(rows_v, out_hbm.at[pl.ds(base, CH)])

            return carry

        lax.fori_loop(0, per_w, body, 0)

    return k(table, idx)


def _tc_body(ef_ref, dt_ref, el_ref, hs_ref, nf_ref,
             wx_ref, wh1_ref, wh2_ref, b1_ref, b2_ref,
             wd1_ref, wd2_ref, bd1_ref, bd2_ref,
             attn_ref, eow_ref, eob_ref, nw_ref, nb_ref,
             fcw_ref, fcb_ref, out_ref, *, nb_nodes):
    blk = nb_nodes * DEG
    f32 = jnp.float32
    ef = ef_ref[...]
    dt = dt_ref[...]
    el = el_ref[...]
    idx_t = jnp.clip(el - 1, 0, T - 1)

    wx = wx_ref[...]
    wh1 = wh1_ref[...]
    wh2 = wh2_ref[...]
    b1 = b1_ref[...]
    b2 = b2_ref[...]
    wd1 = wd1_ref[...]
    wd2 = wd2_ref[...]
    bd1 = bd1_ref[...]
    bd2 = bd2_ref[...]

    zeros = jnp.zeros((blk, H), f32)
    h1 = zeros
    c1 = zeros
    h2 = zeros
    c2 = zeros
    h1_sel = zeros
    h2_sel = zeros
    log_e = jnp.log(f32(np.e) + dt)

    def dot(a, b):
        return jnp.dot(a, b, preferred_element_type=f32)

    for t in range(T):
        x_t = ef[:, t * EDGE_IN:(t + 1) * EDGE_IN]
        zx = dot(x_t, wx)
        g = 1.0 / log_e[:, t:t + 1]
        sel = (idx_t == t).astype(f32)

        cs1 = jnp.tanh(dot(c1, wd1) + bd1)
        cadj1 = c1 - cs1 + cs1 * g
        z1 = zx[:, :4 * H] + dot(h1, wh1) + b1
        i1 = z1[:, 0:H]
        f1 = z1[:, H:2 * H]
        o1 = z1[:, 2 * H:3 * H]
        u1 = z1[:, 3 * H:4 * H]
        c1 = jax.nn.sigmoid(f1) * cadj1 + jax.nn.sigmoid(i1) * jnp.tanh(u1)
        h1 = jax.nn.sigmoid(o1) * jnp.tanh(c1)
        h1_sel = h1_sel + h1 * sel

        cs2 = jnp.tanh(dot(c2, wd2) + bd2)
        cadj2 = c2 - cs2 + cs2 * g
        z2 = zx[:, 4 * H:] + dot(h2, wh2) + b2
        i2 = z2[:, 0:H]
        f2 = z2[:, H:2 * H]
        o2 = z2[:, 2 * H:3 * H]
        u2 = z2[:, 3 * H:4 * H]
        c2 = jax.nn.sigmoid(f2) * cadj2 + jax.nn.sigmoid(i2) * jnp.tanh(u2)
        h2 = jax.nn.sigmoid(o2) * jnp.tanh(c2)
        h2_sel = h2_sel + h2 * sel

    e_out = h1_sel
    a = dot(h2_sel, attn_ref[...])
    a = jnp.where(a > 0, a, 0.01 * a)

    eow = eow_ref[...]
    eob = eob_ref[...]
    m = dot(hs_ref[...], eow[:H]) + dot(e_out, eow[H:]) + eob
    m = jnp.maximum(m, 0.0)

    a2 = a.reshape(nb_nodes, DEG)
    z = a2 - jnp.max(a2, axis=-1, keepdims=True)
    zi = z[:, :, None]
    zj = z[:, None, :]
    jj = lax.broadcasted_iota(jnp.int32, (nb_nodes, DEG, DEG), 2)
    ii = lax.broadcasted_iota(jnp.int32, (nb_nodes, DEG, DEG), 1)
    beq = ((zj > zi) | ((zj == zi) & (jj <= ii))).astype(f32)
    p_pos = jnp.sum(beq, axis=2)
    csum = jnp.sum(beq * zj, axis=2)
    isgt = (1.0 + p_pos * z > csum).astype(f32)
    k_sup = jnp.max(isgt * p_pos, axis=-1, keepdims=True)
    s_sup = jnp.sum(isgt * z, axis=-1, keepdims=True)
    tau = (s_sup - 1.0) / k_sup
    alpha = jnp.maximum(z - tau, 0.0)

    m3 = m.reshape(nb_nodes, DEG, H)
    hagg = jnp.sum(m3 * alpha[:, :, None], axis=1)

    selfh = nf_ref[...]
    qself = dot(selfh, eow[:H]) + eob
    hr = hagg - qself
    nw = nw_ref[...]
    act = jnp.maximum(dot(selfh, nw[:H]) + dot(hr, nw[H:]) + nb_ref[...], 0.0)
    out_ref[...] = dot(act, fcw_ref[...]) + fcb_ref[...]


def _const_spec(shape):
    return pl.BlockSpec(shape, lambda i: (0,) * len(shape))


def kernel(node_features, src_idx, edge_features, delta_t, edge_len, params):
    p = params
    n_nodes = node_features.shape[0]
    e_edges = src_idx.shape[0]
    nb = _NB
    blk = nb * DEG
    grid = n_nodes // nb

    hsrc = _sc_gather(node_features, src_idx)

    ef2 = edge_features.reshape(e_edges, T * EDGE_IN)
    el2 = edge_len.reshape(e_edges, 1)
    wx = jnp.concatenate([p["lstm1_Wx"], p["lstm2_Wx"]], axis=1)
    b1 = p["lstm1_b"].reshape(1, 4 * H)
    b2 = p["lstm2_b"].reshape(1, 4 * H)
    bd1 = p["lstm1_bd"].reshape(1, H)
    bd2 = p["lstm2_bd"].reshape(1, H)
    eob = p["eo_b"].reshape(1, H)
    nodeb = p["node_b"].reshape(1, H)
    fcb = p["fc_b"].reshape(1, NUM_CLASS)

    out = pl.pallas_call(
        functools.partial(_tc_body, nb_nodes=nb),
        grid=(grid,),
        in_specs=[
            pl.BlockSpec((blk, T * EDGE_IN), lambda i: (i, 0)),
            pl.BlockSpec((blk, T), lambda i: (i, 0)),
            pl.BlockSpec((blk, 1), lambda i: (i, 0)),
            pl.BlockSpec((blk, H), lambda i: (i, 0)),
            pl.BlockSpec((nb, H), lambda i: (i, 0)),
            _const_spec((EDGE_IN, 8 * H)),
            _const_spec((H, 4 * H)),
            _const_spec((H, 4 * H)),
            _const_spec((1, 4 * H)),
            _const_spec((1, 4 * H)),
            _const_spec((H, H)),
            _const_spec((H, H)),
            _const_spec((1, H)),
            _const_spec((1, H)),
            _const_spec((H, 1)),
            _const_spec((2 * H, H)),
            _const_spec((1, H)),
            _const_spec((2 * H, H)),
            _const_spec((1, H)),
            _const_spec((H, NUM_CLASS)),
            _const_spec((1, NUM_CLASS)),
        ],
        out_specs=pl.BlockSpec((nb, NUM_CLASS), lambda i: (i, 0)),
        out_shape=jax.ShapeDtypeStruct((n_nodes, NUM_CLASS), jnp.float32),
    )(ef2, delta_t, el2, hsrc, node_features,
      wx, p["lstm1_Wh"], p["lstm2_Wh"], b1, b2,
      p["lstm1_Wd"], p["lstm2_Wd"], bd1, bd2,
      p["attn_W"], p["eo_W"], eob, p["node_W"], nodeb,
      p["fc_W"], fcb)
    return out


# speedup vs baseline: 7.6762x; 1.0045x over previous
import functools
import math

import jax
import jax.numpy as jnp
import numpy as np
from jax import lax
from jax.experimental import pallas as pl
from jax.experimental.pallas import tpu as pltpu
from jax.experimental.pallas import tpu_sc as plsc

H = 128
T = 4
EDGE_IN = 16
DEG = 16
NUM_CLASS = 16

_NB = 200


def _sc_gather(table, idx):
    E = idx.shape[0]
    D = table.shape[1]
    CH = 128
    n_chunks = E // CH
    assert n_chunks * CH == E
    info = plsc.get_sparse_core_info()
    nc = info.num_cores
    nw = nc * info.num_subcores
    per_w = math.ceil(n_chunks / nw)
    mesh = plsc.VectorSubcoreMesh(core_axis_name="c", subcore_axis_name="s")

    @functools.partial(
        pl.kernel,
        out_type=jax.ShapeDtypeStruct((E, D), table.dtype),
        mesh=mesh,
        scratch_types=[
            pltpu.VMEM((CH,), jnp.int32),
            pltpu.VMEM((CH, D), jnp.float32),
            pltpu.SemaphoreType.DMA,
        ],
    )
    def k(table_hbm, idx_hbm, out_hbm, idx_v, rows_v, sem):
        wid = lax.axis_index("s") * nc + lax.axis_index("c")

        def body(t, carry):
            c = wid + t * nw

            @pl.when(c < n_chunks)
            def _():
                base = c * CH
                pltpu.sync_copy(idx_hbm.at[pl.ds(base, CH)], idx_v)
                pltpu.async_copy(table_hbm.at[idx_v], rows_v, sem).wait()
                pltpu.sync_copy(rows_v, out_hbm.at[pl.ds(base, CH)])

            return carry

        lax.fori_loop(0, per_w, body, 0)

    return k(table, idx)


def _tc_body(ef_ref, dt_ref, el_ref, hs_ref, nf_ref,
             wx_ref, wh1_ref, wh2_ref, b1_ref, b2_ref,
             wd1_ref, wd2_ref, bd1_ref, bd2_ref,
             attn_ref, eow_ref, eob_ref, nw_ref, nb_ref,
             fcw_ref, fcb_ref, out_ref, *, nb_nodes):
    blk = nb_nodes * DEG
    f32 = jnp.float32
    ef = ef_ref[...]
    dt = dt_ref[...]
    el = el_ref[...]
    idx_t = jnp.clip(el - 1, 0, T - 1)

    wx = wx_ref[...]
    wh1 = wh1_ref[...]
    wh2 = wh2_ref[...]
    b1 = b1_ref[...]
    b2 = b2_ref[...]
    wd1 = wd1_ref[...]
    wd2 = wd2_ref[...]
    bd1 = bd1_ref[...]
    bd2 = bd2_ref[...]

    zeros = jnp.zeros((blk, H), f32)
    h1 = zeros
    c1 = zeros
    h2 = zeros
    c2 = zeros
    h1_sel = zeros
    h2_sel = zeros
    log_e = jnp.log(f32(np.e) + dt)

    def dot(a, b):
        bf = jnp.bfloat16
        return jnp.dot(a.astype(bf), b.astype(bf), preferred_element_type=f32)

    for t in range(T):
        x_t = ef[:, t * EDGE_IN:(t + 1) * EDGE_IN]
        zx = dot(x_t, wx)
        g = 1.0 / log_e[:, t:t + 1]
        sel = (idx_t == t).astype(f32)

        cs1 = jnp.tanh(dot(c1, wd1) + bd1)
        cadj1 = c1 - cs1 + cs1 * g
        z1 = zx[:, :4 * H] + dot(h1, wh1) + b1
        i1 = z1[:, 0:H]
        f1 = z1[:, H:2 * H]
        o1 = z1[:, 2 * H:3 * H]
        u1 = z1[:, 3 * H:4 * H]
        c1 = jax.nn.sigmoid(f1) * cadj1 + jax.nn.sigmoid(i1) * jnp.tanh(u1)
        h1 = jax.nn.sigmoid(o1) * jnp.tanh(c1)
        h1_sel = h1_sel + h1 * sel

        cs2 = jnp.tanh(dot(c2, wd2) + bd2)
        cadj2 = c2 - cs2 + cs2 * g
        z2 = zx[:, 4 * H:] + dot(h2, wh2) + b2
        i2 = z2[:, 0:H]
        f2 = z2[:, H:2 * H]
        o2 = z2[:, 2 * H:3 * H]
        u2 = z2[:, 3 * H:4 * H]
        c2 = jax.nn.sigmoid(f2) * cadj2 + jax.nn.sigmoid(i2) * jnp.tanh(u2)
        h2 = jax.nn.sigmoid(o2) * jnp.tanh(c2)
        h2_sel = h2_sel + h2 * sel

    e_out = h1_sel
    a = dot(h2_sel, attn_ref[...])
    a = jnp.where(a > 0, a, 0.01 * a)

    eow = eow_ref[...]
    eob = eob_ref[...]
    m = dot(hs_ref[...], eow[:H]) + dot(e_out, eow[H:]) + eob
    m = jnp.maximum(m, 0.0)

    a2 = a.reshape(nb_nodes, DEG)
    z = a2 - jnp.max(a2, axis=-1, keepdims=True)
    zi = z[:, :, None]
    zj = z[:, None, :]
    jj = lax.broadcasted_iota(jnp.int32, (nb_nodes, DEG, DEG), 2)
    ii = lax.broadcasted_iota(jnp.int32, (nb_nodes, DEG, DEG), 1)
    beq = ((zj > zi) | ((zj == zi) & (jj <= ii))).astype(f32)
    p_pos = jnp.sum(beq, axis=2)
    csum = jnp.sum(beq * zj, axis=2)
    isgt = (1.0 + p_pos * z > csum).astype(f32)
    k_sup = jnp.max(isgt * p_pos, axis=-1, keepdims=True)
    s_sup = jnp.sum(isgt * z, axis=-1, keepdims=True)
    tau = (s_sup - 1.0) / k_sup
    alpha = jnp.maximum(z - tau, 0.0)

    m3 = m.reshape(nb_nodes, DEG, H)
    hagg = jnp.sum(m3 * alpha[:, :, None], axis=1)

    selfh = nf_ref[...]
    qself = dot(selfh, eow[:H]) + eob
    hr = hagg - qself
    nw = nw_ref[...]
    act = jnp.maximum(dot(selfh, nw[:H]) + dot(hr, nw[H:]) + nb_ref[...], 0.0)
    out_ref[...] = dot(act, fcw_ref[...]) + fcb_ref[...]


def _const_spec(shape):
    return pl.BlockSpec(shape, lambda i: (0,) * len(shape))


def kernel(node_features, src_idx, edge_features, delta_t, edge_len, params):
    p = params
    n_nodes = node_features.shape[0]
    e_edges = src_idx.shape[0]
    nb = _NB
    blk = nb * DEG
    grid = n_nodes // nb

    hsrc = _sc_gather(node_features, src_idx)

    ef2 = edge_features.reshape(e_edges, T * EDGE_IN)
    el2 = edge_len.reshape(e_edges, 1)
    wx = jnp.concatenate([p["lstm1_Wx"], p["lstm2_Wx"]], axis=1)
    b1 = p["lstm1_b"].reshape(1, 4 * H)
    b2 = p["lstm2_b"].reshape(1, 4 * H)
    bd1 = p["lstm1_bd"].reshape(1, H)
    bd2 = p["lstm2_bd"].reshape(1, H)
    eob = p["eo_b"].reshape(1, H)
    nodeb = p["node_b"].reshape(1, H)
    fcb = p["fc_b"].reshape(1, NUM_CLASS)

    out = pl.pallas_call(
        functools.partial(_tc_body, nb_nodes=nb),
        grid=(grid,),
        in_specs=[
            pl.BlockSpec((blk, T * EDGE_IN), lambda i: (i, 0)),
            pl.BlockSpec((blk, T), lambda i: (i, 0)),
            pl.BlockSpec((blk, 1), lambda i: (i, 0)),
            pl.BlockSpec((blk, H), lambda i: (i, 0)),
            pl.BlockSpec((nb, H), lambda i: (i, 0)),
            _const_spec((EDGE_IN, 8 * H)),
            _const_spec((H, 4 * H)),
            _const_spec((H, 4 * H)),
            _const_spec((1, 4 * H)),
            _const_spec((1, 4 * H)),
            _const_spec((H, H)),
            _const_spec((H, H)),
            _const_spec((1, H)),
            _const_spec((1, H)),
            _const_spec((H, 1)),
            _const_spec((2 * H, H)),
            _const_spec((1, H)),
            _const_spec((2 * H, H)),
            _const_spec((1, H)),
            _const_spec((H, NUM_CLASS)),
            _const_spec((1, NUM_CLASS)),
        ],
        out_specs=pl.BlockSpec((nb, NUM_CLASS), lambda i: (i, 0)),
        out_shape=jax.ShapeDtypeStruct((n_nodes, NUM_CLASS), jnp.float32),
    )(ef2, delta_t, el2, hsrc, node_features,
      wx, p["lstm1_Wh"], p["lstm2_Wh"], b1, b2,
      p["lstm1_Wd"], p["lstm2_Wd"], bd1, bd2,
      p["attn_W"], p["eo_W"], eob, p["node_W"], nodeb,
      p["fc_W"], fcb)
    return out


# speedup vs baseline: 8.5758x; 1.1172x over previous
import functools
import math

import jax
import jax.numpy as jnp
import numpy as np
from jax import lax
from jax.experimental import pallas as pl
from jax.experimental.pallas import tpu as pltpu
from jax.experimental.pallas import tpu_sc as plsc

H = 128
T = 4
EDGE_IN = 16
DEG = 16
NUM_CLASS = 16

_NB = 200
_NB_POST = 200


def _sc_gather(table, idx):
    E = idx.shape[0]
    D = table.shape[1]
    CH = 128
    n_chunks = E // CH
    assert n_chunks * CH == E
    info = plsc.get_sparse_core_info()
    nc = info.num_cores
    nw = nc * info.num_subcores
    per_w = math.ceil(n_chunks / nw)
    mesh = plsc.VectorSubcoreMesh(core_axis_name="c", subcore_axis_name="s")

    @functools.partial(
        pl.kernel,
        out_type=jax.ShapeDtypeStruct((E, D), table.dtype),
        mesh=mesh,
        scratch_types=[
            pltpu.VMEM((CH,), jnp.int32),
            pltpu.VMEM((CH, D), jnp.float32),
            pltpu.SemaphoreType.DMA,
        ],
    )
    def k(table_hbm, idx_hbm, out_hbm, idx_v, rows_v, sem):
        wid = lax.axis_index("s") * nc + lax.axis_index("c")

        def body(t, carry):
            c = wid + t * nw

            @pl.when(c < n_chunks)
            def _():
                base = c * CH
                pltpu.sync_copy(idx_hbm.at[pl.ds(base, CH)], idx_v)
                pltpu.async_copy(table_hbm.at[idx_v], rows_v, sem).wait()
                pltpu.sync_copy(rows_v, out_hbm.at[pl.ds(base, CH)])

            return carry

        lax.fori_loop(0, per_w, body, 0)

    return k(table, idx)


def _dot(a, b):
    bf = jnp.bfloat16
    return jnp.dot(a.astype(bf), b.astype(bf), preferred_element_type=jnp.float32)


def _q_body(nf_ref, w_ref, b_ref, q_ref):
    q_ref[...] = _dot(nf_ref[...], w_ref[...]) + b_ref[...]


def _lstm_body(ef_ref, dt_ref, el_ref,
               wx_ref, wh1_ref, wh2_ref, b1_ref, b2_ref,
               wd1_ref, wd2_ref, bd1_ref, bd2_ref,
               attn_ref, eow2_ref,
               me_ref, a_ref, *, blk):
    f32 = jnp.float32
    ef = ef_ref[...]
    dt = dt_ref[...]
    el = el_ref[...]
    idx_t = jnp.clip(el - 1, 0, T - 1)

    wx = wx_ref[...]
    wh1 = wh1_ref[...]
    wh2 = wh2_ref[...]
    b1 = b1_ref[...]
    b2 = b2_ref[...]
    wd1 = wd1_ref[...]
    wd2 = wd2_ref[...]
    bd1 = bd1_ref[...]
    bd2 = bd2_ref[...]

    zeros = jnp.zeros((blk, H), f32)
    h1 = zeros
    c1 = zeros
    h2 = zeros
    c2 = zeros
    h1_sel = zeros
    h2_sel = zeros
    g_all = 1.0 / jnp.log(f32(np.e) + dt)

    for t in range(T):
        x_t = ef[:, t * EDGE_IN:(t + 1) * EDGE_IN]
        zx = _dot(x_t, wx)
        g = g_all[:, t:t + 1]
        sel = (idx_t == t).astype(f32)

        cs1 = jnp.tanh(_dot(c1, wd1) + bd1)
        cadj1 = c1 - cs1 + cs1 * g
        z1 = zx[:, :4 * H] + _dot(h1, wh1) + b1
        c1 = (jax.nn.sigmoid(z1[:, H:2 * H]) * cadj1
              + jax.nn.sigmoid(z1[:, 0:H]) * jnp.tanh(z1[:, 3 * H:]))
        h1 = jax.nn.sigmoid(z1[:, 2 * H:3 * H]) * jnp.tanh(c1)
        h1_sel = h1_sel + h1 * sel

        cs2 = jnp.tanh(_dot(c2, wd2) + bd2)
        cadj2 = c2 - cs2 + cs2 * g
        z2 = zx[:, 4 * H:] + _dot(h2, wh2) + b2
        c2 = (jax.nn.sigmoid(z2[:, H:2 * H]) * cadj2
              + jax.nn.sigmoid(z2[:, 0:H]) * jnp.tanh(z2[:, 3 * H:]))
        h2 = jax.nn.sigmoid(z2[:, 2 * H:3 * H]) * jnp.tanh(c2)
        h2_sel = h2_sel + h2 * sel

    me_ref[...] = _dot(h1_sel, eow2_ref[...])
    a = _dot(h2_sel, attn_ref[...])
    a_ref[...] = jnp.where(a > 0, a, 0.01 * a)


def _post_body(qs_ref, me_ref, a_ref, q_ref, nf_ref,
               nw_ref, nb_ref, fcw_ref, fcb_ref, out_ref, *, nb_nodes):
    f32 = jnp.float32
    m = jnp.maximum(qs_ref[...] + me_ref[...], 0.0)
    a = a_ref[...]

    a2 = a.reshape(nb_nodes, DEG)
    z = a2 - jnp.max(a2, axis=-1, keepdims=True)
    zi = z[:, :, None]
    zj = z[:, None, :]
    jj = lax.broadcasted_iota(jnp.int32, (nb_nodes, DEG, DEG), 2)
    ii = lax.broadcasted_iota(jnp.int32, (nb_nodes, DEG, DEG), 1)
    beq = ((zj > zi) | ((zj == zi) & (jj <= ii))).astype(f32)
    p_pos = jnp.sum(beq, axis=2)
    csum = jnp.sum(beq * zj, axis=2)
    isgt = (1.0 + p_pos * z > csum).astype(f32)
    k_sup = jnp.max(isgt * p_pos, axis=-1, keepdims=True)
    s_sup = jnp.sum(isgt * z, axis=-1, keepdims=True)
    tau = (s_sup - 1.0) / k_sup
    alpha = jnp.maximum(z - tau, 0.0)

    m3 = m.reshape(nb_nodes, DEG, H)
    hagg = jnp.sum(m3 * alpha[:, :, None], axis=1)

    hr = hagg - q_ref[...]
    nw = nw_ref[...]
    act = jnp.maximum(
        _dot(nf_ref[...], nw[:H]) + _dot(hr, nw[H:]) + nb_ref[...], 0.0)
    out_ref[...] = _dot(act, fcw_ref[...]) + fcb_ref[...]


def _const_spec(shape):
    return pl.BlockSpec(shape, lambda i: (0,) * len(shape))


def kernel(node_features, src_idx, edge_features, delta_t, edge_len, params):
    p = params
    n_nodes = node_features.shape[0]
    e_edges = src_idx.shape[0]

    eob = p["eo_b"].reshape(1, H)
    eow = p["eo_W"]

    nb_q = 2000
    q = pl.pallas_call(
        _q_body,
        grid=(n_nodes // nb_q,),
        in_specs=[
            pl.BlockSpec((nb_q, H), lambda i: (i, 0)),
            _const_spec((H, H)),
            _const_spec((1, H)),
        ],
        out_specs=pl.BlockSpec((nb_q, H), lambda i: (i, 0)),
        out_shape=jax.ShapeDtypeStruct((n_nodes, H), jnp.float32),
    )(node_features, eow[:H], eob)

    qsrc = _sc_gather(q, src_idx)

    nb = _NB
    blk = nb * DEG
    ef2 = edge_features.reshape(e_edges, T * EDGE_IN)
    el2 = edge_len.reshape(e_edges, 1)
    wx = jnp.concatenate([p["lstm1_Wx"], p["lstm2_Wx"]], axis=1)
    b1 = p["lstm1_b"].reshape(1, 4 * H)
    b2 = p["lstm2_b"].reshape(1, 4 * H)
    bd1 = p["lstm1_bd"].reshape(1, H)
    bd2 = p["lstm2_bd"].reshape(1, H)

    me, a = pl.pallas_call(
        functools.partial(_lstm_body, blk=blk),
        grid=(e_edges // blk,),
        in_specs=[
            pl.BlockSpec((blk, T * EDGE_IN), lambda i: (i, 0)),
            pl.BlockSpec((blk, T), lambda i: (i, 0)),
            pl.BlockSpec((blk, 1), lambda i: (i, 0)),
            _const_spec((EDGE_IN, 8 * H)),
            _const_spec((H, 4 * H)),
            _const_spec((H, 4 * H)),
            _const_spec((1, 4 * H)),
            _const_spec((1, 4 * H)),
            _const_spec((H, H)),
            _const_spec((H, H)),
            _const_spec((1, H)),
            _const_spec((1, H)),
            _const_spec((H, 1)),
            _const_spec((H, H)),
        ],
        out_specs=[
            pl.BlockSpec((blk, H), lambda i: (i, 0)),
            pl.BlockSpec((blk, 1), lambda i: (i, 0)),
        ],
        out_shape=[
            jax.ShapeDtypeStruct((e_edges, H), jnp.float32),
            jax.ShapeDtypeStruct((e_edges, 1), jnp.float32),
        ],
    )(ef2, delta_t, el2,
      wx, p["lstm1_Wh"], p["lstm2_Wh"], b1, b2,
      p["lstm1_Wd"], p["lstm2_Wd"], bd1, bd2,
      p["attn_W"], eow[H:])

    nbp = _NB_POST
    blkp = nbp * DEG
    nodeb = p["node_b"].reshape(1, H)
    fcb = p["fc_b"].reshape(1, NUM_CLASS)
    out = pl.pallas_call(
        functools.partial(_post_body, nb_nodes=nbp),
        grid=(n_nodes // nbp,),
        in_specs=[
            pl.BlockSpec((blkp, H), lambda i: (i, 0)),
            pl.BlockSpec((blkp, H), lambda i: (i, 0)),
            pl.BlockSpec((blkp, 1), lambda i: (i, 0)),
            pl.BlockSpec((nbp, H), lambda i: (i, 0)),
            pl.BlockSpec((nbp, H), lambda i: (i, 0)),
            _const_spec((2 * H, H)),
            _const_spec((1, H)),
            _const_spec((H, NUM_CLASS)),
            _const_spec((1, NUM_CLASS)),
        ],
        out_specs=pl.BlockSpec((nbp, NUM_CLASS), lambda i: (i, 0)),
        out_shape=jax.ShapeDtypeStruct((n_nodes, NUM_CLASS), jnp.float32),
    )(qsrc, me, a, q, node_features,
      p["node_W"], nodeb, p["fc_W"], fcb)
    return out
